# baseline (speedup 1.0000x reference)
import functools

import jax
import jax.numpy as jnp
from jax import lax
from jax.experimental import pallas as pl
from jax.experimental.pallas import tpu as pltpu
from jax.experimental.pallas import tpu_sc as plsc


def _harmgram_sc(spec_flat, idx_flat, R, K, F):
    info = plsc.get_sparse_core_info()
    nc, ns = info.num_cores, info.num_subcores
    nw = nc * ns
    rows_w = R // nw
    kv = K // 16

    mesh = plsc.VectorSubcoreMesh(core_axis_name="c", subcore_axis_name="s")

    nk_w = rows_w * K

    @functools.partial(
        pl.kernel,
        mesh=mesh,
        out_type=jax.ShapeDtypeStruct((R * K,), jnp.float32),
        scratch_types=[
            pltpu.VMEM((K,), jnp.int32),
            pltpu.VMEM((nk_w,), jnp.int32),
            pltpu.VMEM((nk_w,), jnp.float32),
            pltpu.SemaphoreType.DMA,
        ],
    )
    def run(spec_hbm, idx_hbm, out_hbm, base_v, idx_v, data_v, sem):
        wid = lax.axis_index("s") * nc + lax.axis_index("c")
        row0 = wid * rows_w
        pltpu.sync_copy(idx_hbm, base_v)

        off0 = row0 * F
        carry = [base_v[pl.ds(16 * j, 16)] + off0 for j in range(kv)]

        def body(r, c):
            for j in range(kv):
                idx_v[pl.ds(r * K + 16 * j, 16)] = c[j]
            return [v + F for v in c]

        lax.fori_loop(0, rows_w, body, carry)

        pltpu.async_copy(spec_hbm.at[idx_v], data_v, sem).wait()
        pltpu.sync_copy(data_v, out_hbm.at[pl.ds(row0 * K, nk_w)])

    return run(spec_flat, idx_flat)


def kernel(specgram, hargram_idx):
    B, T, F = specgram.shape
    P, H = hargram_idx.shape
    spec_flat = specgram.reshape(-1)
    idx_flat = hargram_idx.reshape(-1).astype(jnp.int32)
    out = _harmgram_sc(spec_flat, idx_flat, B * T, P * H, F)
    return out.reshape(B, T, P, H)


# speedup vs baseline: 1.0051x; 1.0051x over previous
import functools

import jax
import jax.numpy as jnp
from jax import lax
from jax.experimental import pallas as pl
from jax.experimental.pallas import tpu as pltpu
from jax.experimental.pallas import tpu_sc as plsc


def _harmgram_sc(spec_flat, idx_flat, R, K, F):
    info = plsc.get_sparse_core_info()
    nc, ns = info.num_cores, info.num_subcores
    nw = nc * ns
    rows_w = R // nw
    kv = K // 16

    mesh = plsc.VectorSubcoreMesh(core_axis_name="c", subcore_axis_name="s")

    nk_w = rows_w * K
    nch = 10
    rows_ch = rows_w // nch
    nk_ch = rows_ch * K

    @functools.partial(
        pl.kernel,
        mesh=mesh,
        out_type=jax.ShapeDtypeStruct((R * K,), jnp.float32),
        scratch_types=[
            pltpu.VMEM((K,), jnp.int32),
            pltpu.VMEM((nk_w,), jnp.int32),
            pltpu.VMEM((nk_w,), jnp.float32),
            pltpu.SemaphoreType.DMA,
        ],
    )
    def run(spec_hbm, idx_hbm, out_hbm, base_v, idx_v, data_v, sem):
        wid = lax.axis_index("s") * nc + lax.axis_index("c")
        row0 = wid * rows_w
        pltpu.sync_copy(idx_hbm, base_v)

        off0 = row0 * F
        carry = [base_v[pl.ds(16 * j, 16)] + off0 for j in range(kv)]

        copies = []
        for ch in range(nch):
            base_el = ch * nk_ch

            def body(r, c, base_el=base_el):
                for j in range(kv):
                    idx_v[pl.ds(base_el + r * K + 16 * j, 16)] = c[j]
                return [v + F for v in c]

            carry = lax.fori_loop(0, rows_ch, body, carry)
            copies.append(
                pltpu.async_copy(
                    spec_hbm.at[idx_v.at[pl.ds(base_el, nk_ch)]],
                    data_v.at[pl.ds(base_el, nk_ch)],
                    sem,
                )
            )
        for cp in copies:
            cp.wait()
        pltpu.sync_copy(data_v, out_hbm.at[pl.ds(row0 * K, nk_w)])

    return run(spec_flat, idx_flat)


def kernel(specgram, hargram_idx):
    B, T, F = specgram.shape
    P, H = hargram_idx.shape
    spec_flat = specgram.reshape(-1)
    idx_flat = hargram_idx.reshape(-1).astype(jnp.int32)
    out = _harmgram_sc(spec_flat, idx_flat, B * T, P * H, F)
    return out.reshape(B, T, P, H)
